# baseline (speedup 1.0000x reference)
import functools

import jax
import jax.numpy as jnp
from jax import lax
from jax.experimental import pallas as pl
from jax.experimental.pallas import tpu as pltpu
from jax.experimental.pallas import tpu_sc as plsc

N = 10000
D = 128
DOUT = 64
E = 320000

NC = 2
NS = 16
NW = NC * NS
CH = 112
NCH = 90
EW = NCH * CH
EP = NW * EW
NPAD = 10112
RPT = NPAD // NS
NB = 3
NPAD_DEG = 10240
RPT_DEG = NPAD_DEG // NS

_mesh = plsc.VectorSubcoreMesh(
    core_axis_name="c", subcore_axis_name="s", num_cores=NC, num_subcores=NS)


def _worker_id():
  return lax.axis_index("s") * NC + lax.axis_index("c")


def _unpack_chunk(pbuf, j, sstage, dstage):
  @pl.loop(0, CH // 16)
  def _(k):
    v = pbuf[j, pl.ds(k * 16, 16)]
    sstage[pl.ds(k * 16, 16)] = jnp.bitwise_and(v, 0xFFFF)
    dstage[pl.ds(k * 16, 16)] = lax.shift_right_logical(v, 16)


@functools.partial(
    pl.kernel,
    out_type=jax.ShapeDtypeStruct((NC, NPAD_DEG), jnp.float32),
    mesh=_mesh,
    scratch_types=[
        pltpu.VMEM((NCH, CH), jnp.int32),
        pltpu.VMEM((CH,), jnp.int32),
        pltpu.VMEM((CH,), jnp.int32),
        pltpu.VMEM((CH,), jnp.float32),
        pltpu.VMEM((RPT_DEG,), jnp.float32),
        pltpu.VMEM_SHARED((NPAD_DEG,), jnp.float32),
    ],
)
def _deg_sc(pidx_hbm, out_hbm, pbuf, sstage, dstage, ones, zbuf, acc):
  cid = lax.axis_index("c")
  sid = lax.axis_index("s")
  wid = _worker_id()

  onev = jnp.ones((16,), jnp.float32)
  zerov = jnp.zeros((16,), jnp.float32)

  @pl.loop(0, CH // 16)
  def _(i):
    ones[pl.ds(i * 16, 16)] = onev

  @pl.loop(0, RPT_DEG // 16)
  def _(i):
    zbuf[pl.ds(i * 16, 16)] = zerov

  pltpu.sync_copy(pidx_hbm.at[wid], pbuf)
  pltpu.sync_copy(zbuf, acc.at[pl.ds(sid * RPT_DEG, RPT_DEG)])
  plsc.subcore_barrier()

  @pl.loop(0, NCH)
  def _(c):
    _unpack_chunk(pbuf, c, sstage, dstage)
    pltpu.sync_copy(ones, acc.at[dstage], add=True)

  plsc.subcore_barrier()
  pltpu.sync_copy(acc.at[pl.ds(sid * RPT_DEG, RPT_DEG)],
                  out_hbm.at[cid, pl.ds(sid * RPT_DEG, RPT_DEG)])


_spmm_scratch = (
    [pltpu.VMEM((CH,), jnp.int32) for _ in range(NB)]
    + [pltpu.VMEM((CH,), jnp.int32) for _ in range(NB)]
    + [pltpu.VMEM((CH,), jnp.int32) for _ in range(NB)]
    + [pltpu.VMEM((NB, CH, D), jnp.float32)]
    + [pltpu.VMEM_SHARED((NPAD, D), jnp.float32)]
    + [pltpu.SemaphoreType.DMA for _ in range(3 * NB)]
)

_STRIPES = []
_off = 0
while _off < RPT:
  _STRIPES.append((_off, min(CH, RPT - _off)))
  _off += CH


@functools.partial(
    pl.kernel,
    out_type=jax.ShapeDtypeStruct((NC, NPAD, D), jnp.float32),
    mesh=_mesh,
    scratch_types=_spmm_scratch,
)
def _spmm_sc(h_hbm, pidx_hbm, out_hbm, *rest):
  ib = rest[0:NB]
  ss = rest[NB:2 * NB]
  dd = rest[2 * NB:3 * NB]
  gbuf = rest[3 * NB]
  acc = rest[3 * NB + 1]
  isem = rest[3 * NB + 2:3 * NB + 2 + NB]
  gsem = rest[3 * NB + 2 + NB:3 * NB + 2 + 2 * NB]
  ssem = rest[3 * NB + 2 + 2 * NB:3 * NB + 2 + 3 * NB]

  cid = lax.axis_index("c")
  sid = lax.axis_index("s")
  wid = _worker_id()

  zerov = jnp.zeros((16,), jnp.float32)

  @pl.loop(0, CH)
  def _(r):
    @pl.loop(0, D // 16)
    def _(k):
      gbuf[0, r, pl.ds(k * 16, 16)] = zerov

  for off, rows in _STRIPES:
    pltpu.sync_copy(gbuf.at[0, pl.ds(0, rows)],
                    acc.at[pl.ds(sid * RPT + off, rows)])

  plsc.subcore_barrier()

  def _icopy(j, b):
    return pltpu.make_async_copy(pidx_hbm.at[wid, j], ib[b], isem[b])

  def _unpack_src(b):
    @pl.loop(0, CH // 16)
    def _(k):
      v = ib[b][pl.ds(k * 16, 16)]
      ss[b][pl.ds(k * 16, 16)] = jnp.bitwise_and(v, 0xFFFF)

  def _unpack_dst(b):
    @pl.loop(0, CH // 16)
    def _(k):
      v = ib[b][pl.ds(k * 16, 16)]
      dd[b][pl.ds(k * 16, 16)] = lax.shift_right_logical(v, 16)

  def _gather(b):
    return pltpu.make_async_copy(h_hbm.at[ss[b]], gbuf.at[b], gsem[b])

  def _scatter(b):
    return pltpu.make_async_copy(gbuf.at[b], acc.at[dd[b]], ssem[b])

  for b in range(NB):
    _icopy(b, b).start()
  _icopy(0, 0).wait()
  _unpack_src(0)
  _gather(0).start()

  @pl.loop(0, NCH, step=NB)
  def _(c):
    for b in range(NB):
      j = c + b
      b1 = (b + 1) % NB
      _gather(b).wait()
      _unpack_dst(b)
      pltpu.async_copy(gbuf.at[b], acc.at[dd[b]], ssem[b], add=True)

      @pl.when(j + NB < NCH)
      def _():
        _icopy(j + NB, b).start()

      @pl.when(j + 1 < NCH)
      def _():
        @pl.when(j >= 2)
        def _():
          _scatter(b1).wait()

        _icopy(j + 1, b1).wait()
        _unpack_src(b1)
        _gather(b1).start()

  for b in range(NB):
    _scatter((NCH - NB + 1 + b) % NB).wait()

  plsc.subcore_barrier()

  for off, rows in _STRIPES:
    row = sid * RPT + off
    pltpu.sync_copy(acc.at[pl.ds(row, rows)],
                    out_hbm.at[cid, pl.ds(row, rows)])


def _tc1_body(p_ref, x_ref, w_ref, hp_ref, dinv_ref):
  deg = 1.0 + p_ref[0, :N] + p_ref[1, :N]
  dinv = lax.rsqrt(deg)
  dinv_ref[...] = dinv
  h = jnp.dot(x_ref[...], w_ref[...], preferred_element_type=jnp.float32)
  hp_ref[...] = h * dinv[:, None]


_tc1 = pl.pallas_call(
    _tc1_body,
    out_shape=[
        jax.ShapeDtypeStruct((N, D), jnp.float32),
        jax.ShapeDtypeStruct((N,), jnp.float32),
    ],
)


def _tc_mid_body(p_ref, hp_ref, dinv_ref, b_ref, w_ref, out_ref):
  dinv = dinv_ref[...]
  agg = p_ref[0, :N] + p_ref[1, :N] + hp_ref[...]
  z = jnp.maximum(agg * dinv[:, None] + b_ref[...], 0.0)
  h = jnp.dot(z, w_ref[...], preferred_element_type=jnp.float32)
  out_ref[...] = h * dinv[:, None]


_tc_mid = pl.pallas_call(
    _tc_mid_body,
    out_shape=jax.ShapeDtypeStruct((N, D), jnp.float32),
)


def _tc_final_body(p_ref, hp_ref, dinv_ref, b_ref, wout_ref, bout_ref,
                   node_ref, graph_ref):
  dinv = dinv_ref[...]
  agg = p_ref[0, :N] + p_ref[1, :N] + hp_ref[...]
  z = jnp.maximum(agg * dinv[:, None] + b_ref[...], 0.0)
  node = jnp.dot(z, wout_ref[...], preferred_element_type=jnp.float32)
  node = node + bout_ref[...]
  node_ref[...] = node
  graph_ref[...] = jnp.mean(node, axis=0, keepdims=True)


_tc_final = pl.pallas_call(
    _tc_final_body,
    out_shape=[
        jax.ShapeDtypeStruct((N, DOUT), jnp.float32),
        jax.ShapeDtypeStruct((1, DOUT), jnp.float32),
    ],
)


def kernel(x, edge_index, W1, b1, W2, b2, W3, b3, Wout, bout):
  npad = EP - E
  pad_src = jnp.arange(npad, dtype=jnp.int32) % N
  pad_dst = N + jnp.arange(npad, dtype=jnp.int32) % (NPAD - N)
  sidx = jnp.concatenate([edge_index[0], pad_src])
  didx = jnp.concatenate([edge_index[1], pad_dst])
  pidx = (sidx | (didx << 16)).reshape(EW, NW).T.reshape(NW, NCH, CH)

  degp = _deg_sc(pidx)
  h1p, dinv = _tc1(degp, x, W1)
  p = _spmm_sc(h1p, pidx)
  h2p = _tc_mid(p, h1p, dinv, b1, W2)
  p = _spmm_sc(h2p, pidx)
  h3p = _tc_mid(p, h2p, dinv, b2, W3)
  p = _spmm_sc(h3p, pidx)
  node_preds, graph_preds = _tc_final(p, h3p, dinv, b3, Wout, bout)
  return node_preds, graph_preds


# speedup vs baseline: 1.2942x; 1.2942x over previous
import functools

import jax
import jax.numpy as jnp
from jax import lax
from jax.experimental import pallas as pl
from jax.experimental.pallas import tpu as pltpu
from jax.experimental.pallas import tpu_sc as plsc

N = 10000
D = 128
DOUT = 64
E = 320000

NC = 2
NS = 16
NW = NC * NS
CH = 96
NCH = 105
EW = NCH * CH
EP = NW * EW
NPAD = 10112
RPT = NPAD // NS
NB = 3
NPAD_DEG = 10240
RPT_DEG = NPAD_DEG // NS

_mesh = plsc.VectorSubcoreMesh(
    core_axis_name="c", subcore_axis_name="s", num_cores=NC, num_subcores=NS)


def _worker_id():
  return lax.axis_index("s") * NC + lax.axis_index("c")


def _unpack_chunk(pbuf, j, sstage, dstage):
  @pl.loop(0, CH // 16)
  def _(k):
    v = pbuf[j, pl.ds(k * 16, 16)]
    sstage[pl.ds(k * 16, 16)] = jnp.bitwise_and(v, 0xFFFF)
    dstage[pl.ds(k * 16, 16)] = lax.shift_right_logical(v, 16)


@functools.partial(
    pl.kernel,
    out_type=jax.ShapeDtypeStruct((NC, NPAD_DEG), jnp.float32),
    mesh=_mesh,
    scratch_types=[
        pltpu.VMEM((NCH, CH), jnp.int32),
        pltpu.VMEM((CH,), jnp.int32),
        pltpu.VMEM((CH,), jnp.int32),
        pltpu.VMEM((CH,), jnp.float32),
        pltpu.VMEM((RPT_DEG,), jnp.float32),
        pltpu.VMEM_SHARED((NPAD_DEG,), jnp.float32),
    ],
)
def _deg_sc(pidx_hbm, out_hbm, pbuf, sstage, dstage, ones, zbuf, acc):
  cid = lax.axis_index("c")
  sid = lax.axis_index("s")
  wid = _worker_id()

  onev = jnp.ones((16,), jnp.float32)
  zerov = jnp.zeros((16,), jnp.float32)

  @pl.loop(0, CH // 16)
  def _(i):
    ones[pl.ds(i * 16, 16)] = onev

  @pl.loop(0, RPT_DEG // 16)
  def _(i):
    zbuf[pl.ds(i * 16, 16)] = zerov

  pltpu.sync_copy(pidx_hbm.at[wid], pbuf)
  pltpu.sync_copy(zbuf, acc.at[pl.ds(sid * RPT_DEG, RPT_DEG)])
  plsc.subcore_barrier()

  @pl.loop(0, NCH)
  def _(c):
    _unpack_chunk(pbuf, c, sstage, dstage)
    pltpu.sync_copy(ones, acc.at[dstage], add=True)

  plsc.subcore_barrier()
  pltpu.sync_copy(acc.at[pl.ds(sid * RPT_DEG, RPT_DEG)],
                  out_hbm.at[cid, pl.ds(sid * RPT_DEG, RPT_DEG)])


_spmm_scratch = (
    [pltpu.VMEM((EW,), jnp.int32)]
    + [pltpu.VMEM((CH,), jnp.int32) for _ in range(NB)]
    + [pltpu.VMEM((CH,), jnp.int32) for _ in range(NB)]
    + [pltpu.VMEM((NB, CH, D), jnp.float32)]
    + [pltpu.VMEM_SHARED((NPAD, D), jnp.float32)]
    + [pltpu.SemaphoreType.DMA for _ in range(NB)]
    + [pltpu.SemaphoreType.DMA]
)

_STRIPES = []
_off = 0
while _off < RPT:
  _STRIPES.append((_off, min(CH, RPT - _off)))
  _off += CH


@functools.partial(
    pl.kernel,
    out_type=jax.ShapeDtypeStruct((NC, NPAD, D), jnp.float32),
    mesh=_mesh,
    scratch_types=_spmm_scratch,
)
def _spmm_sc(h_hbm, pidx_hbm, out_hbm, pbuf, *rest):
  ss = rest[0:NB]
  dd = rest[NB:2 * NB]
  gbuf = rest[2 * NB]
  acc = rest[2 * NB + 1]
  gsem = rest[2 * NB + 2:2 * NB + 2 + NB]
  ssem = rest[2 * NB + 2 + NB]

  cid = lax.axis_index("c")
  sid = lax.axis_index("s")
  wid = _worker_id()

  pltpu.sync_copy(pidx_hbm.at[wid], pbuf)

  zerov = jnp.zeros((16,), jnp.float32)

  @pl.loop(0, CH)
  def _(r):
    @pl.loop(0, D // 16)
    def _(k):
      gbuf[0, r, pl.ds(k * 16, 16)] = zerov

  for off, rows in _STRIPES:
    pltpu.sync_copy(gbuf.at[0, pl.ds(0, rows)],
                    acc.at[pl.ds(sid * RPT + off, rows)])

  plsc.subcore_barrier()

  def _unpack_src(j, b):
    @pl.loop(0, CH // 16)
    def _(k):
      v = pbuf[pl.ds(j * CH + k * 16, 16)]
      ss[b][pl.ds(k * 16, 16)] = jnp.bitwise_and(v, 0xFFFF)

  def _unpack_dst(j, b):
    @pl.loop(0, CH // 16)
    def _(k):
      v = pbuf[pl.ds(j * CH + k * 16, 16)]
      dd[b][pl.ds(k * 16, 16)] = lax.shift_right_logical(v, 16)

  def _gather(b):
    return pltpu.make_async_copy(h_hbm.at[ss[b]], gbuf.at[b], gsem[b])

  def _drain_one():
    pltpu.make_async_copy(h_hbm.at[pl.ds(0, CH)], gbuf.at[0], ssem).wait()

  _unpack_src(0, 0)
  _gather(0).start()
  _unpack_src(1, 1)
  _gather(1).start()

  @pl.loop(0, NCH, step=NB)
  def _(c):
    for b in range(NB):
      j = c + b
      b2 = (b + 2) % NB
      _gather(b).wait()
      _unpack_dst(j, b)
      pltpu.async_copy(gbuf.at[b], acc.at[dd[b]], ssem, add=True)

      @pl.when(j + 2 < NCH)
      def _():
        @pl.when(j >= 1)
        def _():
          _drain_one()

        _unpack_src(j + 2, b2)
        _gather(b2).start()

  for _ in range(3):
    _drain_one()

  plsc.subcore_barrier()

  for off, rows in _STRIPES:
    row = sid * RPT + off
    pltpu.sync_copy(acc.at[pl.ds(row, rows)],
                    out_hbm.at[cid, pl.ds(row, rows)])


def _tc1_body(p_ref, x_ref, w_ref, hp_ref, dinv_ref):
  deg = 1.0 + p_ref[0, :N] + p_ref[1, :N]
  dinv = lax.rsqrt(deg)
  dinv_ref[...] = dinv
  h = jnp.dot(x_ref[...], w_ref[...], preferred_element_type=jnp.float32)
  hp_ref[...] = h * dinv[:, None]


_tc1 = pl.pallas_call(
    _tc1_body,
    out_shape=[
        jax.ShapeDtypeStruct((N, D), jnp.float32),
        jax.ShapeDtypeStruct((N,), jnp.float32),
    ],
)


def _tc_mid_body(p_ref, hp_ref, dinv_ref, b_ref, w_ref, out_ref):
  dinv = dinv_ref[...]
  agg = p_ref[0, :N] + p_ref[1, :N] + hp_ref[...]
  z = jnp.maximum(agg * dinv[:, None] + b_ref[...], 0.0)
  h = jnp.dot(z, w_ref[...], preferred_element_type=jnp.float32)
  out_ref[...] = h * dinv[:, None]


_tc_mid = pl.pallas_call(
    _tc_mid_body,
    out_shape=jax.ShapeDtypeStruct((N, D), jnp.float32),
)


def _tc_final_body(p_ref, hp_ref, dinv_ref, b_ref, wout_ref, bout_ref,
                   node_ref, graph_ref):
  dinv = dinv_ref[...]
  agg = p_ref[0, :N] + p_ref[1, :N] + hp_ref[...]
  z = jnp.maximum(agg * dinv[:, None] + b_ref[...], 0.0)
  node = jnp.dot(z, wout_ref[...], preferred_element_type=jnp.float32)
  node = node + bout_ref[...]
  node_ref[...] = node
  graph_ref[...] = jnp.mean(node, axis=0, keepdims=True)


_tc_final = pl.pallas_call(
    _tc_final_body,
    out_shape=[
        jax.ShapeDtypeStruct((N, DOUT), jnp.float32),
        jax.ShapeDtypeStruct((1, DOUT), jnp.float32),
    ],
)


def kernel(x, edge_index, W1, b1, W2, b2, W3, b3, Wout, bout):
  npad = EP - E
  pad_src = jnp.arange(npad, dtype=jnp.int32) % N
  pad_dst = N + jnp.arange(npad, dtype=jnp.int32) % (NPAD - N)
  sidx = jnp.concatenate([edge_index[0], pad_src])
  didx = jnp.concatenate([edge_index[1], pad_dst])
  pidx = (sidx | (didx << 16)).reshape(EW, NW).T

  degp = _deg_sc(pidx.reshape(NW, NCH, CH))
  h1p, dinv = _tc1(degp, x, W1)
  p = _spmm_sc(h1p, pidx)
  h2p = _tc_mid(p, h1p, dinv, b1, W2)
  p = _spmm_sc(h2p, pidx)
  h3p = _tc_mid(p, h2p, dinv, b2, W3)
  p = _spmm_sc(h3p, pidx)
  node_preds, graph_preds = _tc_final(p, h3p, dinv, b3, Wout, bout)
  return node_preds, graph_preds


# speedup vs baseline: 1.3222x; 1.0217x over previous
import functools

import jax
import jax.numpy as jnp
from jax import lax
from jax.experimental import pallas as pl
from jax.experimental.pallas import tpu as pltpu
from jax.experimental.pallas import tpu_sc as plsc

N = 10000
D = 128
DOUT = 64
E = 320000

NC = 2
NS = 16
NW = NC * NS
CH = 96
NCH = 105
EW = NCH * CH
EP = NW * EW
NPAD = 10112
RPT = NPAD // NS
NB = 3
NPAD_DEG = 10240
RPT_DEG = NPAD_DEG // NS

_mesh = plsc.VectorSubcoreMesh(
    core_axis_name="c", subcore_axis_name="s", num_cores=NC, num_subcores=NS)


def _worker_id():
  return lax.axis_index("s") * NC + lax.axis_index("c")


def _unpack_chunk(pbuf, j, sstage, dstage):
  @pl.loop(0, CH // 16)
  def _(k):
    v = pbuf[j, pl.ds(k * 16, 16)]
    sstage[pl.ds(k * 16, 16)] = jnp.bitwise_and(v, 0xFFFF)
    dstage[pl.ds(k * 16, 16)] = lax.shift_right_logical(v, 16)


@functools.partial(
    pl.kernel,
    out_type=jax.ShapeDtypeStruct((NC, NPAD_DEG), jnp.float32),
    mesh=_mesh,
    scratch_types=[
        pltpu.VMEM((NCH, CH), jnp.int32),
        pltpu.VMEM((CH,), jnp.int32),
        pltpu.VMEM((CH,), jnp.int32),
        pltpu.VMEM((CH,), jnp.float32),
        pltpu.VMEM((RPT_DEG,), jnp.float32),
        pltpu.VMEM_SHARED((NPAD_DEG,), jnp.float32),
    ],
)
def _deg_sc(pidx_hbm, out_hbm, pbuf, sstage, dstage, ones, zbuf, acc):
  cid = lax.axis_index("c")
  sid = lax.axis_index("s")
  wid = _worker_id()

  onev = jnp.ones((16,), jnp.float32)
  zerov = jnp.zeros((16,), jnp.float32)

  @pl.loop(0, CH // 16)
  def _(i):
    ones[pl.ds(i * 16, 16)] = onev

  @pl.loop(0, RPT_DEG // 16)
  def _(i):
    zbuf[pl.ds(i * 16, 16)] = zerov

  pltpu.sync_copy(pidx_hbm.at[wid], pbuf)
  pltpu.sync_copy(zbuf, acc.at[pl.ds(sid * RPT_DEG, RPT_DEG)])
  plsc.subcore_barrier()

  @pl.loop(0, NCH)
  def _(c):
    _unpack_chunk(pbuf, c, sstage, dstage)
    pltpu.sync_copy(ones, acc.at[dstage], add=True)

  plsc.subcore_barrier()
  pltpu.sync_copy(acc.at[pl.ds(sid * RPT_DEG, RPT_DEG)],
                  out_hbm.at[cid, pl.ds(sid * RPT_DEG, RPT_DEG)])


_spmm_scratch = (
    [pltpu.VMEM((EW,), jnp.int32)]
    + [pltpu.VMEM((CH,), jnp.int32) for _ in range(NB)]
    + [pltpu.VMEM((CH,), jnp.int32) for _ in range(NB)]
    + [pltpu.VMEM((NB, CH, D), jnp.float32)]
    + [pltpu.VMEM_SHARED((NPAD, D), jnp.float32)]
    + [pltpu.SemaphoreType.DMA for _ in range(NB)]
    + [pltpu.SemaphoreType.DMA]
)

_STRIPES = []
_off = 0
while _off < RPT:
  _STRIPES.append((_off, min(CH, RPT - _off)))
  _off += CH


@functools.partial(
    pl.kernel,
    out_type=jax.ShapeDtypeStruct((NC, NPAD, D), jnp.float32),
    mesh=_mesh,
    scratch_types=_spmm_scratch,
)
def _spmm_sc(h_hbm, pidx_hbm, out_hbm, pbuf, *rest):
  ss = rest[0:NB]
  dd = rest[NB:2 * NB]
  gbuf = rest[2 * NB]
  acc = rest[2 * NB + 1]
  gsem = rest[2 * NB + 2:2 * NB + 2 + NB]
  ssem = rest[2 * NB + 2 + NB]

  cid = lax.axis_index("c")
  sid = lax.axis_index("s")
  wid = _worker_id()

  pltpu.sync_copy(pidx_hbm.at[wid], pbuf)

  zerov = jnp.zeros((16,), jnp.float32)

  @pl.loop(0, CH)
  def _(r):
    @pl.loop(0, D // 16)
    def _(k):
      gbuf[0, r, pl.ds(k * 16, 16)] = zerov

  for off, rows in _STRIPES:
    pltpu.sync_copy(gbuf.at[0, pl.ds(0, rows)],
                    acc.at[pl.ds(sid * RPT + off, rows)])

  plsc.subcore_barrier()

  def _unpack_src(j, b):
    @pl.loop(0, CH // 16)
    def _(k):
      v = pbuf[pl.ds(j * CH + k * 16, 16)]
      ss[b][pl.ds(k * 16, 16)] = jnp.bitwise_and(v, 0xFFFF)

  def _unpack_dst(j, b):
    @pl.loop(0, CH // 16)
    def _(k):
      v = pbuf[pl.ds(j * CH + k * 16, 16)]
      dd[b][pl.ds(k * 16, 16)] = lax.shift_right_logical(v, 16)

  def _gather(b):
    return pltpu.make_async_copy(h_hbm.at[ss[b]], gbuf.at[b], gsem[b])

  def _drain_one():
    pltpu.make_async_copy(h_hbm.at[pl.ds(0, CH)], gbuf.at[0], ssem).wait()

  _unpack_src(0, 0)
  _gather(0).start()
  _unpack_src(1, 1)
  _gather(1).start()

  @pl.loop(0, NCH, step=NB)
  def _(c):
    for b in range(NB):
      j = c + b
      b2 = (b + 2) % NB
      _gather(b).wait()
      _unpack_dst(j, b)
      pltpu.async_copy(gbuf.at[b], acc.at[dd[b]], ssem, add=True)

      @pl.when(j + 2 < NCH)
      def _():
        @pl.when(j >= 1)
        def _():
          _drain_one()

        _unpack_src(j + 2, b2)
        _gather(b2).start()

  for _ in range(3):
    _drain_one()

  plsc.subcore_barrier()

  for off, rows in _STRIPES:
    row = sid * RPT + off
    pltpu.sync_copy(acc.at[pl.ds(row, rows)],
                    out_hbm.at[cid, pl.ds(row, rows)])


def _tc1_body(p_ref, x_ref, w_ref, hp_ref, dinv_ref):
  deg = 1.0 + p_ref[0, :N] + p_ref[1, :N]
  dinv = lax.rsqrt(deg)
  dinv_ref[...] = dinv
  h = jnp.dot(x_ref[...], w_ref[...], preferred_element_type=jnp.float32)
  hp_ref[...] = h * dinv[:, None]


_tc1 = pl.pallas_call(
    _tc1_body,
    out_shape=[
        jax.ShapeDtypeStruct((N, D), jnp.float32),
        jax.ShapeDtypeStruct((N,), jnp.float32),
    ],
)


def _tc_mid_body(p_ref, hp_ref, dinv_ref, b_ref, w_ref, out_ref):
  dinv = dinv_ref[...]
  agg = p_ref[0, :N] + p_ref[1, :N] + hp_ref[...]
  z = jnp.maximum(agg * dinv[:, None] + b_ref[...], 0.0)
  h = jnp.dot(z, w_ref[...], preferred_element_type=jnp.float32)
  out_ref[...] = h * dinv[:, None]


_tc_mid = pl.pallas_call(
    _tc_mid_body,
    out_shape=jax.ShapeDtypeStruct((N, D), jnp.float32),
)


def _tc_final_body(p_ref, hp_ref, dinv_ref, b_ref, wout_ref, bout_ref,
                   node_ref, graph_ref):
  dinv = dinv_ref[...]
  agg = p_ref[0, :N] + p_ref[1, :N] + hp_ref[...]
  z = jnp.maximum(agg * dinv[:, None] + b_ref[...], 0.0)
  node = jnp.dot(z, wout_ref[...], preferred_element_type=jnp.float32)
  node = node + bout_ref[...]
  node_ref[...] = node
  graph_ref[...] = jnp.mean(node, axis=0, keepdims=True)


_tc_final = pl.pallas_call(
    _tc_final_body,
    out_shape=[
        jax.ShapeDtypeStruct((N, DOUT), jnp.float32),
        jax.ShapeDtypeStruct((1, DOUT), jnp.float32),
    ],
)


def kernel(x, edge_index, W1, b1, W2, b2, W3, b3, Wout, bout):
  npad = EP - E
  pad_src = jnp.arange(npad, dtype=jnp.int32) % N
  pad_dst = N + jnp.arange(npad, dtype=jnp.int32) % (NPAD - N)
  sidx = jnp.concatenate([edge_index[0], pad_src])
  didx = jnp.concatenate([edge_index[1], pad_dst])
  pidx = (sidx | (didx << 16)).reshape(NW, EW)

  degp = _deg_sc(pidx.reshape(NW, NCH, CH))
  h1p, dinv = _tc1(degp, x, W1)
  p = _spmm_sc(h1p, pidx)
  h2p = _tc_mid(p, h1p, dinv, b1, W2)
  p = _spmm_sc(h2p, pidx)
  h3p = _tc_mid(p, h2p, dinv, b2, W3)
  p = _spmm_sc(h3p, pidx)
  node_preds, graph_preds = _tc_final(p, h3p, dinv, b3, Wout, bout)
  return node_preds, graph_preds


# speedup vs baseline: 1.3371x; 1.0113x over previous
import functools

import jax
import jax.numpy as jnp
from jax import lax
from jax.experimental import pallas as pl
from jax.experimental.pallas import tpu as pltpu
from jax.experimental.pallas import tpu_sc as plsc

N = 10000
D = 128
DOUT = 64
E = 320000

NC = 2
NS = 16
NW = NC * NS
CH = 96
NCH = 105
EW = NCH * CH
EP = NW * EW
NPAD = 10112
RPT = NPAD // NS
NB = 3
NPAD_DEG = 10240
RPT_DEG = NPAD_DEG // NS

_mesh = plsc.VectorSubcoreMesh(
    core_axis_name="c", subcore_axis_name="s", num_cores=NC, num_subcores=NS)


def _worker_id():
  return lax.axis_index("s") * NC + lax.axis_index("c")


@functools.partial(
    pl.kernel,
    out_type=jax.ShapeDtypeStruct((NC, NPAD_DEG), jnp.float32),
    mesh=_mesh,
    scratch_types=[
        pltpu.VMEM((NCH, CH), jnp.int32),
        pltpu.VMEM((2, CH), jnp.int32),
        pltpu.VMEM((CH,), jnp.float32),
        pltpu.VMEM((RPT_DEG,), jnp.float32),
        pltpu.VMEM_SHARED((NPAD_DEG,), jnp.float32),
        pltpu.SemaphoreType.DMA,
    ],
)
def _deg_sc(pidx_hbm, out_hbm, pbuf, dstage, ones, zbuf, acc, ssem):
  cid = lax.axis_index("c")
  sid = lax.axis_index("s")
  wid = _worker_id()

  onev = jnp.ones((16,), jnp.float32)
  zerov = jnp.zeros((16,), jnp.float32)

  @pl.loop(0, CH // 16)
  def _(i):
    ones[pl.ds(i * 16, 16)] = onev

  @pl.loop(0, RPT_DEG // 16)
  def _(i):
    zbuf[pl.ds(i * 16, 16)] = zerov

  pltpu.sync_copy(pidx_hbm.at[wid], pbuf)
  pltpu.sync_copy(zbuf, acc.at[pl.ds(sid * RPT_DEG, RPT_DEG)])
  plsc.subcore_barrier()

  def _unpack_dst(c, b):
    @pl.loop(0, CH // 16)
    def _(k):
      v = pbuf[c, pl.ds(k * 16, 16)]
      dstage[b, pl.ds(k * 16, 16)] = lax.shift_right_logical(v, 16)

  def _drain_one():
    pltpu.make_async_copy(out_hbm.at[0, pl.ds(0, CH)], ones, ssem).wait()

  @pl.loop(0, NCH - 1, step=2)
  def _(c):
    for b in range(2):
      j = c + b
      @pl.when(j >= 2)
      def _():
        _drain_one()

      _unpack_dst(j, b)
      pltpu.async_copy(ones, acc.at[dstage.at[b]], ssem, add=True)

  _drain_one()
  _unpack_dst(NCH - 1, 0)
  pltpu.async_copy(ones, acc.at[dstage.at[0]], ssem, add=True)
  _drain_one()
  _drain_one()

  plsc.subcore_barrier()
  pltpu.sync_copy(acc.at[pl.ds(sid * RPT_DEG, RPT_DEG)],
                  out_hbm.at[cid, pl.ds(sid * RPT_DEG, RPT_DEG)])


_spmm_scratch = (
    [pltpu.VMEM((EW,), jnp.int32)]
    + [pltpu.VMEM((CH,), jnp.int32) for _ in range(NB)]
    + [pltpu.VMEM((CH,), jnp.int32) for _ in range(NB)]
    + [pltpu.VMEM((NB, CH, D), jnp.float32)]
    + [pltpu.VMEM_SHARED((NPAD, D), jnp.float32)]
    + [pltpu.SemaphoreType.DMA for _ in range(NB)]
    + [pltpu.SemaphoreType.DMA]
)

_STRIPES = []
_off = 0
while _off < RPT:
  _STRIPES.append((_off, min(CH, RPT - _off)))
  _off += CH


@functools.partial(
    pl.kernel,
    out_type=jax.ShapeDtypeStruct((NC, NPAD, D), jnp.float32),
    mesh=_mesh,
    scratch_types=_spmm_scratch,
)
def _spmm_sc(h_hbm, pidx_hbm, out_hbm, pbuf, *rest):
  ss = rest[0:NB]
  dd = rest[NB:2 * NB]
  gbuf = rest[2 * NB]
  acc = rest[2 * NB + 1]
  gsem = rest[2 * NB + 2:2 * NB + 2 + NB]
  ssem = rest[2 * NB + 2 + NB]

  cid = lax.axis_index("c")
  sid = lax.axis_index("s")
  wid = _worker_id()

  pltpu.sync_copy(pidx_hbm.at[wid], pbuf)

  zerov = jnp.zeros((16,), jnp.float32)

  @pl.loop(0, CH)
  def _(r):
    @pl.loop(0, D // 16)
    def _(k):
      gbuf[0, r, pl.ds(k * 16, 16)] = zerov

  for off, rows in _STRIPES:
    pltpu.sync_copy(gbuf.at[0, pl.ds(0, rows)],
                    acc.at[pl.ds(sid * RPT + off, rows)])

  plsc.subcore_barrier()

  def _unpack_src(j, b):
    @pl.loop(0, CH // 16)
    def _(k):
      v = pbuf[pl.ds(j * CH + k * 16, 16)]
      ss[b][pl.ds(k * 16, 16)] = jnp.bitwise_and(v, 0xFFFF)

  def _unpack_dst(j, b):
    @pl.loop(0, CH // 16)
    def _(k):
      v = pbuf[pl.ds(j * CH + k * 16, 16)]
      dd[b][pl.ds(k * 16, 16)] = lax.shift_right_logical(v, 16)

  def _gather(b):
    return pltpu.make_async_copy(h_hbm.at[ss[b]], gbuf.at[b], gsem[b])

  def _drain_one():
    pltpu.make_async_copy(h_hbm.at[pl.ds(0, CH)], gbuf.at[0], ssem).wait()

  _unpack_src(0, 0)
  _gather(0).start()
  _unpack_src(1, 1)
  _gather(1).start()

  @pl.loop(0, NCH, step=NB)
  def _(c):
    for b in range(NB):
      j = c + b
      b2 = (b + 2) % NB
      _gather(b).wait()
      _unpack_dst(j, b)
      pltpu.async_copy(gbuf.at[b], acc.at[dd[b]], ssem, add=True)

      @pl.when(j + 2 < NCH)
      def _():
        @pl.when(j >= 1)
        def _():
          _drain_one()

        _unpack_src(j + 2, b2)
        _gather(b2).start()

  for _ in range(3):
    _drain_one()

  plsc.subcore_barrier()

  for off, rows in _STRIPES:
    row = sid * RPT + off
    pltpu.sync_copy(acc.at[pl.ds(row, rows)],
                    out_hbm.at[cid, pl.ds(row, rows)])


def _tc1_body(p_ref, x_ref, w_ref, hp_ref, dinv_ref):
  deg = 1.0 + p_ref[0, :N] + p_ref[1, :N]
  dinv = lax.rsqrt(deg)
  dinv_ref[...] = dinv
  h = jnp.dot(x_ref[...], w_ref[...], preferred_element_type=jnp.float32)
  hp_ref[...] = h * dinv[:, None]


_tc1 = pl.pallas_call(
    _tc1_body,
    out_shape=[
        jax.ShapeDtypeStruct((N, D), jnp.float32),
        jax.ShapeDtypeStruct((N,), jnp.float32),
    ],
)


def _tc_mid_body(p_ref, hp_ref, dinv_ref, b_ref, w_ref, out_ref):
  dinv = dinv_ref[...]
  agg = p_ref[0, :N] + p_ref[1, :N] + hp_ref[...]
  z = jnp.maximum(agg * dinv[:, None] + b_ref[...], 0.0)
  h = jnp.dot(z, w_ref[...], preferred_element_type=jnp.float32)
  out_ref[...] = h * dinv[:, None]


_tc_mid = pl.pallas_call(
    _tc_mid_body,
    out_shape=jax.ShapeDtypeStruct((N, D), jnp.float32),
)


def _tc_final_body(p_ref, hp_ref, dinv_ref, b_ref, wout_ref, bout_ref,
                   node_ref, graph_ref):
  dinv = dinv_ref[...]
  agg = p_ref[0, :N] + p_ref[1, :N] + hp_ref[...]
  z = jnp.maximum(agg * dinv[:, None] + b_ref[...], 0.0)
  node = jnp.dot(z, wout_ref[...], preferred_element_type=jnp.float32)
  node = node + bout_ref[...]
  node_ref[...] = node
  graph_ref[...] = jnp.mean(node, axis=0, keepdims=True)


_tc_final = pl.pallas_call(
    _tc_final_body,
    out_shape=[
        jax.ShapeDtypeStruct((N, DOUT), jnp.float32),
        jax.ShapeDtypeStruct((1, DOUT), jnp.float32),
    ],
)


def kernel(x, edge_index, W1, b1, W2, b2, W3, b3, Wout, bout):
  npad = EP - E
  pad_src = jnp.arange(npad, dtype=jnp.int32) % N
  pad_dst = N + jnp.arange(npad, dtype=jnp.int32) % (NPAD - N)
  sidx = jnp.concatenate([edge_index[0], pad_src])
  didx = jnp.concatenate([edge_index[1], pad_dst])
  pidx = (sidx | (didx << 16)).reshape(NW, EW)

  degp = _deg_sc(pidx.reshape(NW, NCH, CH))
  h1p, dinv = _tc1(degp, x, W1)
  p = _spmm_sc(h1p, pidx)
  h2p = _tc_mid(p, h1p, dinv, b1, W2)
  p = _spmm_sc(h2p, pidx)
  h3p = _tc_mid(p, h2p, dinv, b2, W3)
  p = _spmm_sc(h3p, pidx)
  node_preds, graph_preds = _tc_final(p, h3p, dinv, b3, Wout, bout)
  return node_preds, graph_preds
